# baseline (device time: 135819 ns/iter reference)
import jax
import jax.numpy as jnp
from jax import lax
from jax.experimental import pallas as pl
from jax.experimental.pallas import tpu as pltpu

N_DEV = 8
M_PER = 512
K = 4096
N_PER = 256


def _ring(p):
    return jnp.where(p < 4, p, 11 - p)


def kernel(x, w_mat):
    x16 = x.astype(jnp.bfloat16)
    w16 = w_mat.astype(jnp.bfloat16)

    H = M_PER // 2
    SEG = 128
    RT = 160
    TAIL = H - RT

    def body(x_ref, w_ref, out_ref, gather, w_vmem, amax_src, amax_buf,
             fsend, frecv, bsend, brecv, csend, crecv,
             a_send_sems, a_recv_sems, w_dma_sem):
        me = lax.axis_index("i")
        r = _ring(me)
        sq = jnp.where(r < 4, 0, 4)
        rr = r - sq
        right = _ring(sq + (rr + 1) % 4)
        left = _ring(sq + (rr - 1) % 4)
        zpair = _ring(7 - r)

        barrier = pltpu.get_barrier_semaphore()
        for nbr in (left, right, zpair):
            pl.semaphore_signal(barrier, 1, device_id=(nbr,),
                                device_id_type=pl.DeviceIdType.MESH)
        pl.semaphore_wait(barrier, 3)

        w_dma = pltpu.make_async_copy(w_ref, w_vmem, w_dma_sem)

        m = [_ring(sq + (rr - k) % 4) for k in range(4)]
        p = [_ring(7 - (sq + (rr - k) % 4)) for k in range(4)]
        mb = [_ring(sq + (rr + k) % 4) for k in range(4)]
        pb = [_ring(7 - (sq + (rr + k) % 4)) for k in range(4)]

        ft = [m[0], m[1], m[2], p[0], p[1]]
        fr = [m[1], m[2], m[3], p[1], p[2]]
        bt = [mb[0], mb[1], mb[2], pb[0], pb[1]]
        br = [mb[1], mb[2], mb[3], pb[1], pb[2]]

        def seg_desc(chunk, row0, nrows, ssems, rsems, t, s, dev,
                     from_x=False):
            src_ref = (x_ref.at[pl.ds(row0, nrows)] if from_x
                       else gather.at[chunk, pl.ds(row0, nrows)])
            return pltpu.make_async_remote_copy(
                src_ref=src_ref,
                dst_ref=gather.at[chunk, pl.ds(row0, nrows)],
                send_sem=ssems.at[t, s],
                recv_sem=rsems.at[t, s],
                device_id=(dev,),
                device_id_type=pl.DeviceIdType.MESH,
            )

        sends = []

        def fgo(t, s, row0=None, nrows=SEG, from_x=False):
            row0 = s * SEG if row0 is None else row0
            d = seg_desc(ft[t], row0, nrows, fsend, frecv, t, s, right,
                         from_x)
            d.start()
            sends.append(d)

        def bgo(t, s, row0=None, nrows=SEG, from_x=False):
            row0 = H + s * SEG if row0 is None else row0
            d = seg_desc(bt[t], row0, nrows, bsend, brecv, t, s, left,
                         from_x)
            d.start()
            sends.append(d)

        def fwait(t, s, row0=None, nrows=SEG):
            row0 = s * SEG if row0 is None else row0
            seg_desc(fr[t], row0, nrows, fsend, frecv, t, s, right).wait_recv()

        def bwait(t, s, row0=None, nrows=SEG):
            row0 = H + s * SEG if row0 is None else row0
            seg_desc(br[t], row0, nrows, bsend, brecv, t, s, left).wait_recv()

        def cross_desc(chunk, row0, t, nrows=SEG, from_x=False):
            src_ref = (x_ref.at[pl.ds(row0, nrows)] if from_x
                       else gather.at[chunk, pl.ds(row0, nrows)])
            return pltpu.make_async_remote_copy(
                src_ref=src_ref,
                dst_ref=gather.at[chunk, pl.ds(row0, nrows)],
                send_sem=csend.at[t],
                recv_sem=crecv.at[t],
                device_id=(zpair,),
                device_id_type=pl.DeviceIdType.MESH,
            )

        def cgo(chunk, row0, t, nrows=SEG, from_x=False):
            d = cross_desc(chunk, row0, t, nrows, from_x)
            d.start()
            sends.append(d)

        amax_blocks = []

        def gemm_rows(o, row0, nrows):
            blk = gather[o, pl.ds(row0, nrows)]
            y = jnp.dot(blk, w_vmem[...], preferred_element_type=jnp.float32)
            y = jnp.maximum(y, 0.0)
            amax_blocks.append(jnp.max(y))
            out_ref[pl.ds(o * M_PER + row0, nrows), :] = y

        for s in (0, 1):
            fgo(0, s, from_x=True)
        for s in (0, 1):
            bgo(0, s, from_x=True)
        for s in range(4):
            cgo(me, s * SEG, s, from_x=True)
        w_dma.start()
        w_dma.wait()
        y = jnp.dot(x_ref[...], w_vmem[...],
                    preferred_element_type=jnp.float32)
        y = jnp.maximum(y, 0.0)
        amax_blocks.append(jnp.max(y))
        out_ref[pl.ds(me * M_PER, M_PER), :] = y

        for s in (0, 1):
            fwait(0, s)
            fgo(1, s)
            cgo(m[1], s * SEG, 4 + s)
        for s in (0, 1):
            bwait(0, s)
            bgo(1, s)
            cgo(mb[1], H + s * SEG, 6 + s)
        gemm_rows(m[1], 0, H)
        gemm_rows(mb[1], H, H)

        for s in (0, 1):
            cross_desc(zpair, s * SEG, s).wait_recv()
            fgo(3, s)
        for s in (2, 3):
            cross_desc(zpair, s * SEG, s).wait_recv()
            bgo(3, s - 2)
        gemm_rows(zpair, 0, M_PER)

        for s in (0, 1):
            fwait(1, s)
            fgo(2, s)
        cgo(m[2], RT, 8, TAIL)
        for s in (0, 1):
            bwait(1, s)
            bgo(2, s)
        cgo(mb[2], H + RT, 9, TAIL)
        gemm_rows(m[2], 0, H)
        gemm_rows(mb[2], H, H)

        for s in (0, 1):
            fwait(3, s)
        fgo(4, 0, 0, RT)
        for s in (0, 1):
            bwait(3, s)
        bgo(4, 0, H, RT)
        gemm_rows(p[1], 0, H)
        gemm_rows(pb[1], H, H)

        for s in (0, 1):
            fwait(2, s)
        for s in (0, 1):
            bwait(2, s)
        gemm_rows(m[3], 0, H)
        gemm_rows(mb[3], H, H)

        for s in (0, 1):
            cross_desc(p[3], s * SEG, 4 + s).wait_recv()
        gemm_rows(p[3], 0, H)
        for s in (0, 1):
            cross_desc(pb[3], H + s * SEG, 6 + s).wait_recv()
        gemm_rows(pb[3], H, H)

        fwait(4, 0, 0, RT)
        cross_desc(p[2], RT, 8, TAIL).wait_recv()
        gemm_rows(p[2], 0, H)
        bwait(4, 0, H, RT)
        cross_desc(pb[2], H + RT, 9, TAIL).wait_recv()
        gemm_rows(pb[2], H, H)

        for d in sends:
            d.wait_send()

        amax = jnp.max(jnp.stack(amax_blocks))
        amax_src[...] = jnp.full((1, 128), amax, jnp.float32)
        amax_buf[pl.ds(me, 1)] = jnp.full((1, 128), amax, jnp.float32)
        a_sends = []
        for k in range(N_DEV - 1):
            tgt = _ring((r + 1 + k) % N_DEV)
            a = pltpu.make_async_remote_copy(
                src_ref=amax_src,
                dst_ref=amax_buf.at[pl.ds(me, 1)],
                send_sem=a_send_sems.at[k],
                recv_sem=a_recv_sems.at[me],
                device_id=(tgt,),
                device_id_type=pl.DeviceIdType.MESH,
            )
            a.start()
            a_sends.append(a)
        for k in range(N_DEV - 1):
            src_dev = _ring((r + 1 + k) % N_DEV)
            recv = pltpu.make_async_remote_copy(
                src_ref=amax_src,
                dst_ref=amax_buf.at[pl.ds(src_dev, 1)],
                send_sem=a_send_sems.at[k],
                recv_sem=a_recv_sems.at[src_dev],
                device_id=(src_dev,),
                device_id_type=pl.DeviceIdType.MESH,
            )
            recv.wait_recv()
        for a in a_sends:
            a.wait_send()

        amax_g = jnp.max(amax_buf[...])
        scale = amax_g / 127.0
        vals = out_ref[...]
        q = jnp.clip(jnp.round(vals / scale), -127.0, 127.0)
        out_ref[...] = q * scale

    return pl.pallas_call(
        body,
        out_shape=jax.ShapeDtypeStruct((N_DEV * M_PER, N_PER), jnp.float32),
        in_specs=[pl.BlockSpec(memory_space=pltpu.VMEM),
                  pl.BlockSpec(memory_space=pl.ANY)],
        out_specs=pl.BlockSpec(memory_space=pltpu.VMEM),
        scratch_shapes=[
            pltpu.VMEM((N_DEV, M_PER, K), jnp.bfloat16),
            pltpu.VMEM((K, N_PER), jnp.bfloat16),
            pltpu.VMEM((1, 128), jnp.float32),
            pltpu.VMEM((N_DEV, 128), jnp.float32),
            pltpu.SemaphoreType.DMA((5, 2)),
            pltpu.SemaphoreType.DMA((5, 2)),
            pltpu.SemaphoreType.DMA((5, 2)),
            pltpu.SemaphoreType.DMA((5, 2)),
            pltpu.SemaphoreType.DMA((10,)),
            pltpu.SemaphoreType.DMA((10,)),
            pltpu.SemaphoreType.DMA((N_DEV - 1,)),
            pltpu.SemaphoreType.DMA((N_DEV,)),
            pltpu.SemaphoreType.DMA,
        ],
        compiler_params=pltpu.CompilerParams(
            collective_id=0, vmem_limit_bytes=100 * 1024 * 1024),
    )(x16, w16)


# device time: 128525 ns/iter; 1.0568x vs baseline; 1.0568x over previous
import jax
import jax.numpy as jnp
from jax import lax
from jax.experimental import pallas as pl
from jax.experimental.pallas import tpu as pltpu

N_DEV = 8
M_PER = 512
K = 4096
N_PER = 256


def _ring(p):
    return jnp.where(p < 4, p, 11 - p)


def kernel(x, w_mat):
    H = M_PER // 2
    SEG = 128
    RT = 160
    TAIL = H - RT

    def body(x_ref, w_ref, out_ref, gather, w_f32, w_vmem, amax_src,
             amax_buf, fsend, frecv, bsend, brecv, csend, crecv,
             a_send_sems, a_recv_sems, w_dma_sem):
        me = lax.axis_index("i")
        r = _ring(me)
        sq = jnp.where(r < 4, 0, 4)
        rr = r - sq
        right = _ring(sq + (rr + 1) % 4)
        left = _ring(sq + (rr - 1) % 4)
        zpair = _ring(7 - r)

        barrier = pltpu.get_barrier_semaphore()
        for nbr in (left, right, zpair):
            pl.semaphore_signal(barrier, 1, device_id=(nbr,),
                                device_id_type=pl.DeviceIdType.MESH)
        pl.semaphore_wait(barrier, 3)

        gather[me] = x_ref[...].astype(jnp.bfloat16)
        w_dma = pltpu.make_async_copy(w_ref, w_f32, w_dma_sem)

        m = [_ring(sq + (rr - k) % 4) for k in range(4)]
        p = [_ring(7 - (sq + (rr - k) % 4)) for k in range(4)]
        mb = [_ring(sq + (rr + k) % 4) for k in range(4)]
        pb = [_ring(7 - (sq + (rr + k) % 4)) for k in range(4)]

        ft = [m[0], m[1], m[2], p[0], p[1]]
        fr = [m[1], m[2], m[3], p[1], p[2]]
        bt = [mb[0], mb[1], mb[2], pb[0], pb[1]]
        br = [mb[1], mb[2], mb[3], pb[1], pb[2]]

        def seg_desc(chunk, row0, nrows, ssems, rsems, t, s, dev):
            return pltpu.make_async_remote_copy(
                src_ref=gather.at[chunk, pl.ds(row0, nrows)],
                dst_ref=gather.at[chunk, pl.ds(row0, nrows)],
                send_sem=ssems.at[t, s],
                recv_sem=rsems.at[t, s],
                device_id=(dev,),
                device_id_type=pl.DeviceIdType.MESH,
            )

        sends = []

        def fgo(t, s, row0=None, nrows=SEG):
            row0 = s * SEG if row0 is None else row0
            d = seg_desc(ft[t], row0, nrows, fsend, frecv, t, s, right)
            d.start()
            sends.append(d)

        def bgo(t, s, row0=None, nrows=SEG):
            row0 = H + s * SEG if row0 is None else row0
            d = seg_desc(bt[t], row0, nrows, bsend, brecv, t, s, left)
            d.start()
            sends.append(d)

        def fwait(t, s, row0=None, nrows=SEG):
            row0 = s * SEG if row0 is None else row0
            seg_desc(fr[t], row0, nrows, fsend, frecv, t, s, right).wait_recv()

        def bwait(t, s, row0=None, nrows=SEG):
            row0 = H + s * SEG if row0 is None else row0
            seg_desc(br[t], row0, nrows, bsend, brecv, t, s, left).wait_recv()

        def cross_desc(chunk, row0, t, nrows=SEG):
            return pltpu.make_async_remote_copy(
                src_ref=gather.at[chunk, pl.ds(row0, nrows)],
                dst_ref=gather.at[chunk, pl.ds(row0, nrows)],
                send_sem=csend.at[t],
                recv_sem=crecv.at[t],
                device_id=(zpair,),
                device_id_type=pl.DeviceIdType.MESH,
            )

        def cgo(chunk, row0, t, nrows=SEG):
            d = cross_desc(chunk, row0, t, nrows)
            d.start()
            sends.append(d)

        amax_blocks = []

        def gemm_rows(o, row0, nrows):
            blk = gather[o, pl.ds(row0, nrows)]
            y = jnp.dot(blk, w_vmem[...], preferred_element_type=jnp.float32)
            y = jnp.maximum(y, 0.0)
            amax_blocks.append(jnp.max(y))
            out_ref[pl.ds(o * M_PER + row0, nrows), :] = y

        for s in (0, 1):
            fgo(0, s)
        for s in (0, 1):
            bgo(0, s)
        for s in range(4):
            cgo(me, s * SEG, s)
        w_dma.start()
        w_dma.wait()
        w_vmem[...] = w_f32[...].astype(jnp.bfloat16)
        gemm_rows(me, 0, M_PER)

        for s in (0, 1):
            fwait(0, s)
            fgo(1, s)
            cgo(m[1], s * SEG, 4 + s)
        for s in (0, 1):
            bwait(0, s)
            bgo(1, s)
            cgo(mb[1], H + s * SEG, 6 + s)
        gemm_rows(m[1], 0, H)
        gemm_rows(mb[1], H, H)

        for s in (0, 1):
            cross_desc(zpair, s * SEG, s).wait_recv()
            fgo(3, s)
        for s in (2, 3):
            cross_desc(zpair, s * SEG, s).wait_recv()
            bgo(3, s - 2)
        gemm_rows(zpair, 0, M_PER)

        for s in (0, 1):
            fwait(1, s)
            fgo(2, s)
        cgo(m[2], RT, 8, TAIL)
        for s in (0, 1):
            bwait(1, s)
            bgo(2, s)
        cgo(mb[2], H + RT, 9, TAIL)
        gemm_rows(m[2], 0, H)
        gemm_rows(mb[2], H, H)

        for s in (0, 1):
            fwait(3, s)
        fgo(4, 0, 0, RT)
        for s in (0, 1):
            bwait(3, s)
        bgo(4, 0, H, RT)
        gemm_rows(p[1], 0, H)
        gemm_rows(pb[1], H, H)

        for s in (0, 1):
            fwait(2, s)
        for s in (0, 1):
            bwait(2, s)
        gemm_rows(m[3], 0, H)
        gemm_rows(mb[3], H, H)

        for s in (0, 1):
            cross_desc(p[3], s * SEG, 4 + s).wait_recv()
        gemm_rows(p[3], 0, H)
        for s in (0, 1):
            cross_desc(pb[3], H + s * SEG, 6 + s).wait_recv()
        gemm_rows(pb[3], H, H)

        fwait(4, 0, 0, RT)
        cross_desc(p[2], RT, 8, TAIL).wait_recv()
        gemm_rows(p[2], 0, H)
        bwait(4, 0, H, RT)
        cross_desc(pb[2], H + RT, 9, TAIL).wait_recv()
        gemm_rows(pb[2], H, H)

        for d in sends:
            d.wait_send()

        amax = jnp.max(jnp.stack(amax_blocks))
        amax_src[...] = jnp.full((1, 128), amax, jnp.float32)
        amax_buf[pl.ds(me, 1)] = jnp.full((1, 128), amax, jnp.float32)
        a_sends = []
        for k in range(N_DEV - 1):
            tgt = _ring((r + 1 + k) % N_DEV)
            a = pltpu.make_async_remote_copy(
                src_ref=amax_src,
                dst_ref=amax_buf.at[pl.ds(me, 1)],
                send_sem=a_send_sems.at[k],
                recv_sem=a_recv_sems.at[me],
                device_id=(tgt,),
                device_id_type=pl.DeviceIdType.MESH,
            )
            a.start()
            a_sends.append(a)
        for k in range(N_DEV - 1):
            src_dev = _ring((r + 1 + k) % N_DEV)
            recv = pltpu.make_async_remote_copy(
                src_ref=amax_src,
                dst_ref=amax_buf.at[pl.ds(src_dev, 1)],
                send_sem=a_send_sems.at[k],
                recv_sem=a_recv_sems.at[src_dev],
                device_id=(src_dev,),
                device_id_type=pl.DeviceIdType.MESH,
            )
            recv.wait_recv()
        for a in a_sends:
            a.wait_send()

        amax_g = jnp.max(amax_buf[...])
        scale = amax_g / 127.0
        vals = out_ref[...]
        q = jnp.clip(jnp.round(vals / scale), -127.0, 127.0)
        out_ref[...] = q * scale

    return pl.pallas_call(
        body,
        out_shape=jax.ShapeDtypeStruct((N_DEV * M_PER, N_PER), jnp.float32),
        in_specs=[pl.BlockSpec(memory_space=pltpu.VMEM),
                  pl.BlockSpec(memory_space=pl.ANY)],
        out_specs=pl.BlockSpec(memory_space=pltpu.VMEM),
        scratch_shapes=[
            pltpu.VMEM((N_DEV, M_PER, K), jnp.bfloat16),
            pltpu.VMEM((K, N_PER), jnp.float32),
            pltpu.VMEM((K, N_PER), jnp.bfloat16),
            pltpu.VMEM((1, 128), jnp.float32),
            pltpu.VMEM((N_DEV, 128), jnp.float32),
            pltpu.SemaphoreType.DMA((5, 2)),
            pltpu.SemaphoreType.DMA((5, 2)),
            pltpu.SemaphoreType.DMA((5, 2)),
            pltpu.SemaphoreType.DMA((5, 2)),
            pltpu.SemaphoreType.DMA((10,)),
            pltpu.SemaphoreType.DMA((10,)),
            pltpu.SemaphoreType.DMA((N_DEV - 1,)),
            pltpu.SemaphoreType.DMA((N_DEV,)),
            pltpu.SemaphoreType.DMA,
        ],
        compiler_params=pltpu.CompilerParams(
            collective_id=0, vmem_limit_bytes=100 * 1024 * 1024),
    )(x, w_mat)


# device time: 125293 ns/iter; 1.0840x vs baseline; 1.0258x over previous
import jax
import jax.numpy as jnp
from jax import lax
from jax.experimental import pallas as pl
from jax.experimental.pallas import tpu as pltpu

N_DEV = 8
M_PER = 512
K = 4096
N_PER = 256


def _ring(p):
    return jnp.where(p < 4, p, 11 - p)


def kernel(x, w_mat):
    H = M_PER // 2
    SEG = 128
    RT = 160
    TAIL = H - RT

    def body(x_ref, w_ref, out_ref, gather, x_f32, w_f32, w_vmem, amax_src,
             amax_buf, fsend, frecv, bsend, brecv, csend, crecv,
             a_send_sems, a_recv_sems, x_dma_sems, w_dma_sem):
        me = lax.axis_index("i")
        r = _ring(me)
        sq = jnp.where(r < 4, 0, 4)
        rr = r - sq
        right = _ring(sq + (rr + 1) % 4)
        left = _ring(sq + (rr - 1) % 4)
        zpair = _ring(7 - r)

        x_dmas = [
            pltpu.make_async_copy(x_ref.at[pl.ds(s * SEG, SEG)],
                                  x_f32.at[pl.ds(s * SEG, SEG)],
                                  x_dma_sems.at[s])
            for s in range(4)
        ]
        for d in x_dmas:
            d.start()
        w_dma = pltpu.make_async_copy(w_ref, w_f32, w_dma_sem)
        w_dma.start()

        barrier = pltpu.get_barrier_semaphore()
        for nbr in (left, right, zpair):
            pl.semaphore_signal(barrier, 1, device_id=(nbr,),
                                device_id_type=pl.DeviceIdType.MESH)
        pl.semaphore_wait(barrier, 3)

        m = [_ring(sq + (rr - k) % 4) for k in range(4)]
        p = [_ring(7 - (sq + (rr - k) % 4)) for k in range(4)]
        mb = [_ring(sq + (rr + k) % 4) for k in range(4)]
        pb = [_ring(7 - (sq + (rr + k) % 4)) for k in range(4)]

        ft = [m[0], m[1], m[2], p[0], p[1]]
        fr = [m[1], m[2], m[3], p[1], p[2]]
        bt = [mb[0], mb[1], mb[2], pb[0], pb[1]]
        br = [mb[1], mb[2], mb[3], pb[1], pb[2]]

        def seg_desc(chunk, row0, nrows, ssems, rsems, t, s, dev):
            return pltpu.make_async_remote_copy(
                src_ref=gather.at[chunk, pl.ds(row0, nrows)],
                dst_ref=gather.at[chunk, pl.ds(row0, nrows)],
                send_sem=ssems.at[t, s],
                recv_sem=rsems.at[t, s],
                device_id=(dev,),
                device_id_type=pl.DeviceIdType.MESH,
            )

        sends = []

        def fgo(t, s, row0=None, nrows=SEG):
            row0 = s * SEG if row0 is None else row0
            d = seg_desc(ft[t], row0, nrows, fsend, frecv, t, s, right)
            d.start()
            sends.append(d)

        def bgo(t, s, row0=None, nrows=SEG):
            row0 = H + s * SEG if row0 is None else row0
            d = seg_desc(bt[t], row0, nrows, bsend, brecv, t, s, left)
            d.start()
            sends.append(d)

        def fwait(t, s, row0=None, nrows=SEG):
            row0 = s * SEG if row0 is None else row0
            seg_desc(fr[t], row0, nrows, fsend, frecv, t, s, right).wait_recv()

        def bwait(t, s, row0=None, nrows=SEG):
            row0 = H + s * SEG if row0 is None else row0
            seg_desc(br[t], row0, nrows, bsend, brecv, t, s, left).wait_recv()

        def cross_desc(chunk, row0, t, nrows=SEG):
            return pltpu.make_async_remote_copy(
                src_ref=gather.at[chunk, pl.ds(row0, nrows)],
                dst_ref=gather.at[chunk, pl.ds(row0, nrows)],
                send_sem=csend.at[t],
                recv_sem=crecv.at[t],
                device_id=(zpair,),
                device_id_type=pl.DeviceIdType.MESH,
            )

        def cgo(chunk, row0, t, nrows=SEG):
            d = cross_desc(chunk, row0, t, nrows)
            d.start()
            sends.append(d)

        amax_blocks = []

        def gemm_rows(o, row0, nrows):
            blk = gather[o, pl.ds(row0, nrows)]
            y = jnp.dot(blk, w_vmem[...], preferred_element_type=jnp.float32)
            y = jnp.maximum(y, 0.0)
            amax_blocks.append(jnp.max(y))
            out_ref[pl.ds(o * M_PER + row0, nrows), :] = y

        def x_seg_ready(s):
            x_dmas[s].wait()
            gather[me, pl.ds(s * SEG, SEG)] = (
                x_f32[pl.ds(s * SEG, SEG)].astype(jnp.bfloat16))

        x_seg_ready(0)
        fgo(0, 0)
        cgo(me, 0, 0)
        x_seg_ready(2)
        bgo(0, 0)
        cgo(me, 2 * SEG, 2)
        x_seg_ready(1)
        fgo(0, 1)
        cgo(me, SEG, 1)
        x_seg_ready(3)
        bgo(0, 1)
        cgo(me, 3 * SEG, 3)
        w_dma.wait()
        w_vmem[...] = w_f32[...].astype(jnp.bfloat16)
        gemm_rows(me, 0, M_PER)

        for s in (0, 1):
            fwait(0, s)
            fgo(1, s)
            cgo(m[1], s * SEG, 4 + s)
        for s in (0, 1):
            bwait(0, s)
            bgo(1, s)
            cgo(mb[1], H + s * SEG, 6 + s)
        gemm_rows(m[1], 0, H)
        gemm_rows(mb[1], H, H)

        for s in (0, 1):
            cross_desc(zpair, s * SEG, s).wait_recv()
            fgo(3, s)
        for s in (2, 3):
            cross_desc(zpair, s * SEG, s).wait_recv()
            bgo(3, s - 2)
        gemm_rows(zpair, 0, M_PER)

        for s in (0, 1):
            fwait(1, s)
            fgo(2, s)
        cgo(m[2], RT, 8, TAIL)
        for s in (0, 1):
            bwait(1, s)
            bgo(2, s)
        cgo(mb[2], H + RT, 9, TAIL)
        gemm_rows(m[2], 0, H)
        gemm_rows(mb[2], H, H)

        for s in (0, 1):
            fwait(3, s)
        fgo(4, 0, 0, RT)
        for s in (0, 1):
            bwait(3, s)
        bgo(4, 0, H, RT)
        gemm_rows(p[1], 0, H)
        gemm_rows(pb[1], H, H)

        for s in (0, 1):
            fwait(2, s)
        for s in (0, 1):
            bwait(2, s)
        gemm_rows(m[3], 0, H)
        gemm_rows(mb[3], H, H)

        for s in (0, 1):
            cross_desc(p[3], s * SEG, 4 + s).wait_recv()
        gemm_rows(p[3], 0, H)
        for s in (0, 1):
            cross_desc(pb[3], H + s * SEG, 6 + s).wait_recv()
        gemm_rows(pb[3], H, H)

        fwait(4, 0, 0, RT)
        cross_desc(p[2], RT, 8, TAIL).wait_recv()
        gemm_rows(p[2], 0, H)
        bwait(4, 0, H, RT)
        cross_desc(pb[2], H + RT, 9, TAIL).wait_recv()
        gemm_rows(pb[2], H, H)

        for d in sends:
            d.wait_send()

        amax = jnp.max(jnp.stack(amax_blocks))
        amax_src[...] = jnp.full((1, 128), amax, jnp.float32)
        amax_buf[pl.ds(me, 1)] = jnp.full((1, 128), amax, jnp.float32)
        a_sends = []
        for k in range(N_DEV - 1):
            tgt = _ring((r + 1 + k) % N_DEV)
            a = pltpu.make_async_remote_copy(
                src_ref=amax_src,
                dst_ref=amax_buf.at[pl.ds(me, 1)],
                send_sem=a_send_sems.at[k],
                recv_sem=a_recv_sems.at[me],
                device_id=(tgt,),
                device_id_type=pl.DeviceIdType.MESH,
            )
            a.start()
            a_sends.append(a)
        for k in range(N_DEV - 1):
            src_dev = _ring((r + 1 + k) % N_DEV)
            recv = pltpu.make_async_remote_copy(
                src_ref=amax_src,
                dst_ref=amax_buf.at[pl.ds(src_dev, 1)],
                send_sem=a_send_sems.at[k],
                recv_sem=a_recv_sems.at[src_dev],
                device_id=(src_dev,),
                device_id_type=pl.DeviceIdType.MESH,
            )
            recv.wait_recv()
        for a in a_sends:
            a.wait_send()

        amax_g = jnp.max(amax_buf[...])
        scale = amax_g / 127.0
        vals = out_ref[...]
        q = jnp.clip(jnp.round(vals / scale), -127.0, 127.0)
        out_ref[...] = q * scale

    return pl.pallas_call(
        body,
        out_shape=jax.ShapeDtypeStruct((N_DEV * M_PER, N_PER), jnp.float32),
        in_specs=[pl.BlockSpec(memory_space=pl.ANY),
                  pl.BlockSpec(memory_space=pl.ANY)],
        out_specs=pl.BlockSpec(memory_space=pltpu.VMEM),
        scratch_shapes=[
            pltpu.VMEM((N_DEV, M_PER, K), jnp.bfloat16),
            pltpu.VMEM((M_PER, K), jnp.float32),
            pltpu.VMEM((K, N_PER), jnp.float32),
            pltpu.VMEM((K, N_PER), jnp.bfloat16),
            pltpu.VMEM((1, 128), jnp.float32),
            pltpu.VMEM((N_DEV, 128), jnp.float32),
            pltpu.SemaphoreType.DMA((5, 2)),
            pltpu.SemaphoreType.DMA((5, 2)),
            pltpu.SemaphoreType.DMA((5, 2)),
            pltpu.SemaphoreType.DMA((5, 2)),
            pltpu.SemaphoreType.DMA((10,)),
            pltpu.SemaphoreType.DMA((10,)),
            pltpu.SemaphoreType.DMA((N_DEV - 1,)),
            pltpu.SemaphoreType.DMA((N_DEV,)),
            pltpu.SemaphoreType.DMA((4,)),
            pltpu.SemaphoreType.DMA,
        ],
        compiler_params=pltpu.CompilerParams(
            collective_id=0, vmem_limit_bytes=100 * 1024 * 1024),
    )(x, w_mat)
